# baseline (device time: 31737 ns/iter reference)
import jax
import jax.numpy as jnp
from jax import lax
from jax.experimental import pallas as pl
from jax.experimental.pallas import tpu as pltpu

N_DEV = 4
N_TOK = 2048
D_IN = 512
D_OUT = 1024
E_LOCAL = 4
N_EXPERT = 16
CHUNK = N_TOK // N_DEV
N_HALF = 2
HALF = CHUNK // N_HALF
N_MSG = (N_DEV - 1) * N_HALF
QSCALE = 2.5 / 127.0


def kernel(x, router_W, route_idx, expert_W):
    def body(x_ref, rw_ref, idx_ref, ew_hbm, out_hbm,
             stage_ref, recv_ref, ewv_ref, ewb_ref, outv_ref,
             copy_sems, send_sems, recv_sems):
        my_pos = lax.axis_index("i")

        barrier_sem = pltpu.get_barrier_semaphore()
        for k in range(1, N_DEV):
            pl.semaphore_signal(
                barrier_sem, inc=1,
                device_id=((my_pos + k) % N_DEV,),
                device_id_type=pl.DeviceIdType.MESH,
            )

        cp_ew = [
            pltpu.make_async_copy(ew_hbm.at[j], ewv_ref.at[j],
                                  copy_sems.at[N_HALF + j])
            for j in range(E_LOCAL)
        ]
        for j in range(E_LOCAL):
            cp_ew[j].start()

        rw = rw_ref[:, :]

        def gates(row_start, nrows):
            xc = x_ref[pl.ds(row_start, nrows), :]
            scores = jnp.dot(xc, rw, preferred_element_type=jnp.float32)
            m = jnp.max(scores, axis=1, keepdims=True)
            p = jnp.exp(scores - m)
            eid = lax.broadcasted_iota(jnp.int32, (nrows, N_EXPERT), 1)
            i0 = idx_ref[pl.ds(row_start, nrows), 0:1]
            i1 = idx_ref[pl.ds(row_start, nrows), 1:2]
            mask = (eid == i0) | (eid == i1)
            w = jnp.where(mask, p, 0.0)
            w = w / jnp.sum(w, axis=1, keepdims=True)
            return xc, w, eid

        def chip_gate_sum(w, eid, chip):
            return jnp.sum(
                jnp.where(eid // E_LOCAL == chip, w, 0.0),
                axis=1, keepdims=True,
            )

        def partial_rows(row_start, nrows, use_bf16):
            xc, w, eid = gates(row_start, nrows)

            def gate(j):
                return jnp.sum(
                    jnp.where(eid == my_pos * E_LOCAL + j, w, 0.0),
                    axis=1, keepdims=True,
                )

            if use_bf16:
                xg = jnp.concatenate(
                    [(gate(j) * xc).astype(jnp.bfloat16)
                     for j in range(E_LOCAL)],
                    axis=1,
                )
                acc = jnp.dot(xg, ewb_ref[:, :],
                              preferred_element_type=jnp.float32)
            else:
                acc = None
                for j in range(E_LOCAL):
                    cp_ew[j].wait()
                    term = jnp.dot(gate(j) * xc, ewv_ref[j],
                                   preferred_element_type=jnp.float32)
                    acc = term if acc is None else acc + term
            return acc, chip_gate_sum(w, eid, my_pos)

        rdmas = []
        for k in range(N_DEV - 1):
            tgt = (my_pos + 1 + k) % N_DEV
            for h in range(N_HALF):
                s = k * N_HALF + h
                acc, g = partial_rows(tgt * CHUNK + h * HALF, HALF,
                                      use_bf16=(s > 0))
                scale = jnp.where(g > 0.0, g * QSCALE, 1.0)
                q = jnp.clip(jnp.round(acc / scale), -127.0, 127.0)
                stage_ref[s] = q.astype(jnp.int8)
                if s == 0:
                    pl.semaphore_wait(barrier_sem, N_DEV - 1)
                rdma = pltpu.make_async_remote_copy(
                    src_ref=stage_ref.at[s],
                    dst_ref=recv_ref.at[s],
                    send_sem=send_sems.at[s],
                    recv_sem=recv_sems.at[s],
                    device_id=(tgt,),
                    device_id_type=pl.DeviceIdType.MESH,
                )
                rdma.start()
                rdmas.append(rdma)
                if s == 0:
                    ewb_ref[:, :] = ewv_ref[:, :, :].reshape(
                        E_LOCAL * D_IN, D_OUT
                    ).astype(jnp.bfloat16)

        own, _ = partial_rows(my_pos * CHUNK, CHUNK, use_bf16=True)
        outv_ref[:, :] = own
        _, w_own, eid_own = gates(my_pos * CHUNK, CHUNK)

        out_cp = [
            pltpu.make_async_copy(
                outv_ref.at[pl.ds(h * HALF, HALF), :],
                out_hbm.at[pl.ds(h * HALF, HALF), :],
                copy_sems.at[h],
            )
            for h in range(N_HALF)
        ]
        for k in range(N_DEV - 1):
            sender = (my_pos + N_DEV - 1 - k) % N_DEV
            g = chip_gate_sum(w_own, eid_own, sender)
            scale = jnp.where(g > 0.0, g * QSCALE, 1.0)
            for h in range(N_HALF):
                s = k * N_HALF + h
                rdmas[s].wait_recv()
                outv_ref[pl.ds(h * HALF, HALF), :] = (
                    outv_ref[pl.ds(h * HALF, HALF), :]
                    + recv_ref[s].astype(jnp.float32)
                    * scale[h * HALF:(h + 1) * HALF, :]
                )
                if k == N_DEV - 2:
                    out_cp[h].start()

        for h in range(N_HALF):
            out_cp[h].wait()
        for r in rdmas:
            r.wait_send()

    return pl.pallas_call(
        body,
        out_shape=jax.ShapeDtypeStruct((CHUNK, D_OUT), jnp.float32),
        in_specs=[
            pl.BlockSpec(memory_space=pltpu.VMEM),
            pl.BlockSpec(memory_space=pltpu.VMEM),
            pl.BlockSpec(memory_space=pltpu.VMEM),
            pl.BlockSpec(memory_space=pltpu.MemorySpace.HBM),
        ],
        out_specs=pl.BlockSpec(memory_space=pltpu.MemorySpace.HBM),
        scratch_shapes=[
            pltpu.VMEM((N_MSG, HALF, D_OUT), jnp.int8),
            pltpu.VMEM((N_MSG, HALF, D_OUT), jnp.int8),
            pltpu.VMEM((E_LOCAL, D_IN, D_OUT), jnp.float32),
            pltpu.VMEM((E_LOCAL * D_IN, D_OUT), jnp.bfloat16),
            pltpu.VMEM((CHUNK, D_OUT), jnp.float32),
            pltpu.SemaphoreType.DMA((N_HALF + E_LOCAL,)),
            pltpu.SemaphoreType.DMA((N_MSG,)),
            pltpu.SemaphoreType.DMA((N_MSG,)),
        ],
        compiler_params=pltpu.CompilerParams(collective_id=0),
    )(x, router_W, route_idx, expert_W)


# device time: 30763 ns/iter; 1.0317x vs baseline; 1.0317x over previous
import jax
import jax.numpy as jnp
from jax import lax
from jax.experimental import pallas as pl
from jax.experimental.pallas import tpu as pltpu

N_DEV = 4
N_TOK = 2048
D_IN = 512
D_OUT = 1024
E_LOCAL = 4
N_EXPERT = 16
CHUNK = N_TOK // N_DEV
N_HALF = 2
HALF = CHUNK // N_HALF
N_MSG = (N_DEV - 1) * N_HALF
QSCALE = 2.5 / 127.0


def kernel(x, router_W, route_idx, expert_W):
    def body(x_ref, rw_ref, idx_ref, ew_ref, out_ref,
             stage_ref, recv_ref, ewb_ref, send_sems, recv_sems):
        my_pos = lax.axis_index("i")

        barrier_sem = pltpu.get_barrier_semaphore()
        for k in range(1, N_DEV):
            pl.semaphore_signal(
                barrier_sem, inc=1,
                device_id=((my_pos + k) % N_DEV,),
                device_id_type=pl.DeviceIdType.MESH,
            )

        rw = rw_ref[:, :]

        def gates(row_start, nrows):
            xc = x_ref[pl.ds(row_start, nrows), :]
            scores = jnp.dot(xc, rw, preferred_element_type=jnp.float32)
            m = jnp.max(scores, axis=1, keepdims=True)
            p = jnp.exp(scores - m)
            eid = lax.broadcasted_iota(jnp.int32, (nrows, N_EXPERT), 1)
            i0 = idx_ref[pl.ds(row_start, nrows), 0:1]
            i1 = idx_ref[pl.ds(row_start, nrows), 1:2]
            mask = (eid == i0) | (eid == i1)
            w = jnp.where(mask, p, 0.0)
            w = w / jnp.sum(w, axis=1, keepdims=True)
            return xc, w, eid

        def chip_gate_sum(w, eid, chip):
            return jnp.sum(
                jnp.where(eid // E_LOCAL == chip, w, 0.0),
                axis=1, keepdims=True,
            )

        def partial_rows(row_start, nrows, use_bf16):
            xc, w, eid = gates(row_start, nrows)

            def gate(j):
                return jnp.sum(
                    jnp.where(eid == my_pos * E_LOCAL + j, w, 0.0),
                    axis=1, keepdims=True,
                )

            if use_bf16:
                xg = jnp.concatenate(
                    [(gate(j) * xc).astype(jnp.bfloat16)
                     for j in range(E_LOCAL)],
                    axis=1,
                )
                acc = jnp.dot(xg, ewb_ref[:, :],
                              preferred_element_type=jnp.float32)
            else:
                acc = jnp.dot(gate(0) * xc, ew_ref[0],
                              preferred_element_type=jnp.float32)
                for j in range(1, E_LOCAL):
                    acc = acc + jnp.dot(gate(j) * xc, ew_ref[j],
                                        preferred_element_type=jnp.float32)
            return acc, chip_gate_sum(w, eid, my_pos)

        rdmas = []
        for k in range(N_DEV - 1):
            tgt = (my_pos + 1 + k) % N_DEV
            for h in range(N_HALF):
                s = k * N_HALF + h
                acc, g = partial_rows(tgt * CHUNK + h * HALF, HALF,
                                      use_bf16=(s > 0))
                scale = jnp.where(g > 0.0, g * QSCALE, 1.0)
                q = jnp.clip(jnp.round(acc / scale), -127.0, 127.0)
                stage_ref[s] = q.astype(jnp.int8)
                if s == 0:
                    pl.semaphore_wait(barrier_sem, N_DEV - 1)
                rdma = pltpu.make_async_remote_copy(
                    src_ref=stage_ref.at[s],
                    dst_ref=recv_ref.at[s],
                    send_sem=send_sems.at[s],
                    recv_sem=recv_sems.at[s],
                    device_id=(tgt,),
                    device_id_type=pl.DeviceIdType.MESH,
                )
                rdma.start()
                rdmas.append(rdma)
                if s == 0:
                    ewb_ref[:, :] = ew_ref[:, :, :].reshape(
                        E_LOCAL * D_IN, D_OUT
                    ).astype(jnp.bfloat16)

        own, _ = partial_rows(my_pos * CHUNK, CHUNK, use_bf16=True)
        out_ref[:, :] = own
        _, w_own, eid_own = gates(my_pos * CHUNK, CHUNK)

        for k in range(N_DEV - 1):
            sender = (my_pos + N_DEV - 1 - k) % N_DEV
            g = chip_gate_sum(w_own, eid_own, sender)
            scale = jnp.where(g > 0.0, g * QSCALE, 1.0)
            for h in range(N_HALF):
                s = k * N_HALF + h
                rdmas[s].wait_recv()
                out_ref[pl.ds(h * HALF, HALF), :] = (
                    out_ref[pl.ds(h * HALF, HALF), :]
                    + recv_ref[s].astype(jnp.float32)
                    * scale[h * HALF:(h + 1) * HALF, :]
                )

        for r in rdmas:
            r.wait_send()

    return pl.pallas_call(
        body,
        out_shape=jax.ShapeDtypeStruct((CHUNK, D_OUT), jnp.float32),
        in_specs=[
            pl.BlockSpec(memory_space=pltpu.VMEM),
            pl.BlockSpec(memory_space=pltpu.VMEM),
            pl.BlockSpec(memory_space=pltpu.VMEM),
            pl.BlockSpec(memory_space=pltpu.VMEM),
        ],
        out_specs=pl.BlockSpec(memory_space=pltpu.VMEM),
        scratch_shapes=[
            pltpu.VMEM((N_MSG, HALF, D_OUT), jnp.int8),
            pltpu.VMEM((N_MSG, HALF, D_OUT), jnp.int8),
            pltpu.VMEM((E_LOCAL * D_IN, D_OUT), jnp.bfloat16),
            pltpu.SemaphoreType.DMA((N_MSG,)),
            pltpu.SemaphoreType.DMA((N_MSG,)),
        ],
        compiler_params=pltpu.CompilerParams(collective_id=0),
    )(x, router_W, route_idx, expert_W)
